# baseline (speedup 1.0000x reference)
import functools

import jax
import jax.numpy as jnp
import numpy as np
from jax import lax
from jax.experimental import pallas as pl
from jax.experimental.pallas import tpu as pltpu
from jax.experimental.pallas import tpu_sc as plsc

VOCAB = 100000
D = 1024
B = 4
S = 4096
MAX_LEN = 8192

NC = 2
NS = 16
NW = NC * NS
LANES = 16

P_PER_W = S // NW
C = 16
STEPS = P_PER_W // C
ROWS = B * C
KCHUNKS = D // LANES


def _pos_encoding_np():
    pos = np.arange(MAX_LEN, dtype=np.float32)[:, None]
    _2i = np.arange(0, D, 2, dtype=np.float32)
    enc = np.zeros((MAX_LEN, D), dtype=np.float32)
    enc[:, 0::2] = np.sin(pos / 10000 ** (_2i / D))
    enc[:, 1::2] = np.cos(pos / 10000 ** (_2i / D))
    return enc[:S]


def _embed_kernel(x_hbm, table_hbm, pos_hbm, out_hbm, idx_v, rows_v, pos_v, sem):
    wid = lax.axis_index("s") * NC + lax.axis_index("c")
    p_base = wid * P_PER_W

    def step(si, _):
        p0 = p_base + si * C
        for b in range(B):
            pltpu.sync_copy(x_hbm.at[b, pl.ds(p0, C)],
                            idx_v.at[pl.ds(b * C, C)])
        gather = pltpu.async_copy(table_hbm.at[idx_v], rows_v, sem)
        pltpu.sync_copy(pos_hbm.at[pl.ds(p0, C)], pos_v)
        gather.wait()

        def add_j(j, _):
            def add_k(k, _):
                col = pl.ds(k * LANES, LANES)
                pv = pos_v[j, col]
                for b in range(B):
                    r = b * C + j
                    rows_v[r, col] = rows_v[r, col] + pv
                return 0
            lax.fori_loop(0, KCHUNKS, add_k, 0)
            return 0
        lax.fori_loop(0, C, add_j, 0)

        for b in range(B):
            pltpu.sync_copy(rows_v.at[pl.ds(b * C, C)],
                            out_hbm.at[b, pl.ds(p0, C)])
        return 0

    lax.fori_loop(0, STEPS, step, 0)


@jax.jit
def _embed(x, table, pos):
    mesh = plsc.VectorSubcoreMesh(core_axis_name="c", subcore_axis_name="s",
                                  num_cores=NC, num_subcores=NS)
    f = pl.kernel(
        _embed_kernel,
        out_type=jax.ShapeDtypeStruct((B, S, D), jnp.float32),
        mesh=mesh,
        scratch_types=[
            pltpu.VMEM((ROWS,), jnp.int32),
            pltpu.VMEM((ROWS, D), jnp.float32),
            pltpu.VMEM((C, D), jnp.float32),
            pltpu.SemaphoreType.DMA,
        ],
    )
    return f(x, table, pos)


def kernel(x, table):
    pos = jnp.asarray(_pos_encoding_np())
    return _embed(x, table, pos)


# speedup vs baseline: 2.8040x; 2.8040x over previous
import jax
import jax.numpy as jnp
import numpy as np
from jax import lax
from jax.experimental import pallas as pl
from jax.experimental.pallas import tpu as pltpu
from jax.experimental.pallas import tpu_sc as plsc

VOCAB = 100000
D = 1024
B = 4
S = 4096
MAX_LEN = 8192

NC = 2
NS = 16
NW = NC * NS
LANES = 16

P_PER_W = S // NW
C = 8
NCH = P_PER_W // C
ROWS = B * C
KCH = D // LANES


def _pos_encoding_np():
    pos = np.arange(MAX_LEN, dtype=np.float32)[:, None]
    _2i = np.arange(0, D, 2, dtype=np.float32)
    enc = np.zeros((MAX_LEN, D), dtype=np.float32)
    enc[:, 0::2] = np.sin(pos / 10000 ** (_2i / D))
    enc[:, 1::2] = np.cos(pos / 10000 ** (_2i / D))
    return enc[:S]


def _embed_kernel(x_hbm, table_hbm, pos_hbm, out_hbm,
                  idx_all, rows, posb,
                  gsem0, gsem1, psem0, psem1, ssem0, ssem1):
    gsem = (gsem0, gsem1)
    psem = (psem0, psem1)
    ssem = (ssem0, ssem1)

    wid = lax.axis_index("s") * NC + lax.axis_index("c")
    p_base = pl.multiple_of(wid * P_PER_W, P_PER_W)

    for b in range(B):
        pltpu.sync_copy(x_hbm.at[b, pl.ds(p_base, P_PER_W)], idx_all.at[b])

    def issue_chunk(cj, slot):
        o = pl.multiple_of(cj * C, C)
        p0 = pl.multiple_of(p_base + cj * C, C)
        pltpu.async_copy(pos_hbm.at[pl.ds(p0, C)], posb.at[slot], psem[slot])
        for b in range(B):
            pltpu.async_copy(table_hbm.at[idx_all.at[b, pl.ds(o, C)]],
                             rows.at[slot, pl.ds(b * C, C)], gsem[slot])

    def wait_gathers(slot):
        pltpu.make_async_copy(table_hbm.at[pl.ds(0, ROWS)],
                              rows.at[slot], gsem[slot]).wait()

    def wait_pos(slot):
        pltpu.make_async_copy(pos_hbm.at[pl.ds(0, C)],
                              posb.at[slot], psem[slot]).wait()

    def issue_stores(cj, slot):
        p0 = pl.multiple_of(p_base + cj * C, C)
        for b in range(B):
            pltpu.async_copy(rows.at[slot, pl.ds(b * C, C)],
                             out_hbm.at[b, pl.ds(p0, C)], ssem[slot])

    def wait_stores(slot):
        pltpu.make_async_copy(rows.at[slot],
                              out_hbm.at[0, pl.ds(0, ROWS)], ssem[slot]).wait()

    def add_chunk(slot):
        @pl.loop(0, KCH)
        def _k(k):
            col = pl.ds(k * LANES, LANES)
            for j in range(C):
                pv = posb[slot, j, col]
                for b in range(B):
                    r = b * C + j
                    rows[slot, r, col] = rows[slot, r, col] + pv

    issue_chunk(0, 0)

    @pl.loop(0, NCH, step=2)
    def _steps(cj0):
        for h in range(2):
            cj = cj0 + h
            nxt = h ^ 1

            @pl.when(cj + 1 < NCH)
            def _issue_next():
                @pl.when(cj >= 1)
                def _drain_prev():
                    wait_stores(nxt)
                issue_chunk(cj + 1, nxt)

            wait_pos(h)
            wait_gathers(h)
            add_chunk(h)
            issue_stores(cj, h)

    wait_stores(0)
    wait_stores(1)


@jax.jit
def _embed(x, table, pos):
    mesh = plsc.VectorSubcoreMesh(core_axis_name="c", subcore_axis_name="s",
                                  num_cores=NC, num_subcores=NS)
    f = pl.kernel(
        _embed_kernel,
        out_type=jax.ShapeDtypeStruct((B, S, D), jnp.float32),
        mesh=mesh,
        scratch_types=[
            pltpu.VMEM((B, P_PER_W), jnp.int32),
            pltpu.VMEM((2, ROWS, D), jnp.float32),
            pltpu.VMEM((2, C, D), jnp.float32),
            pltpu.SemaphoreType.DMA,
            pltpu.SemaphoreType.DMA,
            pltpu.SemaphoreType.DMA,
            pltpu.SemaphoreType.DMA,
            pltpu.SemaphoreType.DMA,
            pltpu.SemaphoreType.DMA,
        ],
    )
    return f(x, table, pos)


def kernel(x, table):
    pos = jnp.asarray(_pos_encoding_np())
    return _embed(x, table, pos)
